# baseline (speedup 1.0000x reference)
import functools

import jax
import jax.numpy as jnp
from jax import lax
from jax.experimental import pallas as pl
from jax.experimental.pallas import tpu as pltpu
from jax.experimental.pallas import tpu_sc as plsc

BATCH = 4096
SEQ = 200
D = 64
HID = 128
CLS = 128

NC = 2
NS = 16
NW = NC * NS
LANES = 16

C = 2
SEQP = 208
NIV = SEQP // LANES
ROWS_PER_W = BATCH // NW
CHUNKS_PER_W = ROWS_PER_W // C
NCOL = D // LANES


def _pool_body(x_hbm, emb, out, idx_all, rows0, rows1, acc0, acc1,
               sem0, sem1, osem):
    wid = lax.axis_index("s") * NC + lax.axis_index("c")
    base_row = wid * ROWS_PER_W

    zeros = jnp.zeros((LANES,), jnp.int32)
    for b in range(ROWS_PER_W):
        idx_all[b, pl.ds(SEQP - LANES, LANES)] = zeros
    pltpu.sync_copy(x_hbm.at[pl.ds(base_row, ROWS_PER_W)],
                    idx_all.at[:, pl.ds(0, SEQ)])

    def fire(g, buf, sem):
        for e in range(C):
            b = g * C + e
            for k in range(NIV):
                iv = idx_all[b, pl.ds(k * LANES, LANES)]
                pltpu.async_copy(
                    emb.at[iv],
                    buf.at[pl.ds(e * SEQP + k * LANES, LANES)], sem)

    def wait_all(buf, sem):
        pltpu.make_async_copy(emb.at[pl.ds(0, C * SEQP)], buf, sem).wait()

    def accumulate(g, buf, acc):
        for e in range(C):
            base = e * SEQP

            def red_body(r8, accs):
                accs = list(accs)
                for rr in range(8):
                    r = r8 * 8 + rr
                    s = (rr & 1) * NCOL
                    for c in range(NCOL):
                        accs[s + c] = accs[s + c] + buf[
                            base + r, pl.ds(c * LANES, LANES)]
                return tuple(accs)

            accs = lax.fori_loop(
                0, SEQ // 8, red_body,
                tuple(jnp.zeros((LANES,), jnp.float32)
                      for _ in range(2 * NCOL)))
            for c in range(NCOL):
                acc[e, pl.ds(c * LANES, LANES)] = (
                    (accs[c] + accs[NCOL + c]) * (1.0 / SEQ))
        pltpu.async_copy(acc, out.at[pl.ds(base_row + g * C, C)], osem)

    def drain_out(acc):
        pltpu.make_async_copy(emb.at[pl.ds(0, C)], acc, osem).wait()

    fire(0, rows0, sem0)

    def outer(h, carry):
        ga = 2 * h
        gb = 2 * h + 1
        fire(gb, rows1, sem1)
        wait_all(rows0, sem0)
        accumulate(ga, rows0, acc0)

        @pl.when(gb + 1 < CHUNKS_PER_W)
        def _():
            fire(gb + 1, rows0, sem0)

        wait_all(rows1, sem1)
        accumulate(gb, rows1, acc1)
        drain_out(acc0)
        drain_out(acc1)
        return carry

    lax.fori_loop(0, CHUNKS_PER_W // 2, outer, 0)


@jax.jit
def _pool(x, emb):
    mesh = plsc.VectorSubcoreMesh(core_axis_name="c", subcore_axis_name="s")
    return pl.kernel(
        _pool_body,
        out_type=jax.ShapeDtypeStruct((BATCH, D), jnp.float32),
        mesh=mesh,
        compiler_params=pltpu.CompilerParams(use_tc_tiling_on_sc=False),
        scratch_types=[
            pltpu.VMEM((ROWS_PER_W, SEQP), jnp.int32),
            pltpu.VMEM((C * SEQP, D), jnp.float32),
            pltpu.VMEM((C * SEQP, D), jnp.float32),
            pltpu.VMEM((C, D), jnp.float32),
            pltpu.VMEM((C, D), jnp.float32),
            pltpu.SemaphoreType.DMA,
            pltpu.SemaphoreType.DMA,
            pltpu.SemaphoreType.DMA,
        ],
    )(x, emb)


def _mlp_body(p_ref, w1t_ref, b1_ref, w2t_ref, b2_ref, o_ref):
    h = jnp.dot(p_ref[...], w1t_ref[...],
                preferred_element_type=jnp.float32) + b1_ref[...]
    o_ref[...] = jnp.dot(h, w2t_ref[...],
                         preferred_element_type=jnp.float32) + b2_ref[...]


@jax.jit
def _mlp(pooled, W1t, b1, W2t, b2):
    return pl.pallas_call(
        _mlp_body,
        out_shape=jax.ShapeDtypeStruct((BATCH, CLS), jnp.float32),
    )(pooled, W1t, b1, W2t, b2)


def kernel(x, emb, W1, b1, W2, b2):
    pooled = _pool(x, emb)
    return _mlp(pooled, W1.T, b1[None, :], W2.T, b2[None, :])


# speedup vs baseline: 1.5220x; 1.5220x over previous
import functools

import jax
import jax.numpy as jnp
from jax import lax
from jax.experimental import pallas as pl
from jax.experimental.pallas import tpu as pltpu
from jax.experimental.pallas import tpu_sc as plsc

BATCH = 4096
SEQ = 200
D = 64
HID = 128
CLS = 128

NC = 2
NS = 16
NW = NC * NS
LANES = 16

C = 2
G = 100
NG = C * SEQ // G
ROWS_PER_W = BATCH // NW
CHUNKS_PER_W = ROWS_PER_W // C
NCOL = D // LANES

XB = 128


def _xpose_body(xt_ref, oa_ref, ob_ref):
    t = xt_ref[...].T
    oa_ref[...] = t[:, :G]
    ob_ref[...] = t[:, G:]


@jax.jit
def _xpose(xT):
    return pl.pallas_call(
        _xpose_body,
        grid=(BATCH // XB,),
        in_specs=[pl.BlockSpec((SEQ, XB), lambda i: (0, i))],
        out_specs=[pl.BlockSpec((XB, G), lambda i: (i, 0)),
                   pl.BlockSpec((XB, G), lambda i: (i, 0))],
        out_shape=[jax.ShapeDtypeStruct((BATCH, G), jnp.int32),
                   jax.ShapeDtypeStruct((BATCH, G), jnp.int32)],
    )(xT)


def _pool_body(x2a, x2b, emb, out, idx_a, idx_b, rows_v, acc_v, sem):
    wid = lax.axis_index("s") * NC + lax.axis_index("c")

    def chunk_body(g, carry):
        chunk = wid * CHUNKS_PER_W + g
        pltpu.sync_copy(x2a.at[pl.ds(chunk * C, C)], idx_a)
        pltpu.sync_copy(x2b.at[pl.ds(chunk * C, C)], idx_b)
        cps = []
        for e in range(C):
            cps.append(pltpu.async_copy(
                emb.at[idx_a.at[e]],
                rows_v.at[pl.ds(e * SEQ, G)], sem))
            cps.append(pltpu.async_copy(
                emb.at[idx_b.at[e]],
                rows_v.at[pl.ds(e * SEQ + G, G)], sem))
        for cp in cps:
            cp.wait()
        for e in range(C):
            def red_body(r, accs):
                return tuple(
                    accs[c] + rows_v[e * SEQ + r, pl.ds(c * LANES, LANES)]
                    for c in range(NCOL)
                )
            accs = lax.fori_loop(
                0, SEQ, red_body,
                tuple(jnp.zeros((LANES,), jnp.float32)
                      for _ in range(NCOL)))
            for c in range(NCOL):
                acc_v[e, pl.ds(c * LANES, LANES)] = accs[c] * (1.0 / SEQ)
        pltpu.sync_copy(acc_v, out.at[pl.ds(chunk * C, C)])
        return carry

    lax.fori_loop(0, CHUNKS_PER_W, chunk_body, 0)


@jax.jit
def _pool(x2a, x2b, emb):
    mesh = plsc.VectorSubcoreMesh(core_axis_name="c", subcore_axis_name="s")
    return pl.kernel(
        _pool_body,
        out_type=jax.ShapeDtypeStruct((BATCH, D), jnp.float32),
        mesh=mesh,
        compiler_params=pltpu.CompilerParams(use_tc_tiling_on_sc=False),
        scratch_types=[
            pltpu.VMEM((C, G), jnp.int32),
            pltpu.VMEM((C, G), jnp.int32),
            pltpu.VMEM((C * SEQ, D), jnp.float32),
            pltpu.VMEM((C, D), jnp.float32),
            pltpu.SemaphoreType.DMA,
        ],
    )(x2a, x2b, emb)


def _mlp_body(p_ref, w1t_ref, b1_ref, w2t_ref, b2_ref, o_ref):
    h = jnp.dot(p_ref[...], w1t_ref[...],
                preferred_element_type=jnp.float32) + b1_ref[...]
    o_ref[...] = jnp.dot(h, w2t_ref[...],
                         preferred_element_type=jnp.float32) + b2_ref[...]


@jax.jit
def _mlp(pooled, W1t, b1, W2t, b2):
    return pl.pallas_call(
        _mlp_body,
        out_shape=jax.ShapeDtypeStruct((BATCH, CLS), jnp.float32),
    )(pooled, W1t, b1, W2t, b2)


def kernel(x, emb, W1, b1, W2, b2):
    x2a, x2b = _xpose(x.T)
    pooled = _pool(x2a, x2b, emb)
    return _mlp(pooled, W1.T, b1[None, :], W2.T, b2[None, :])


# speedup vs baseline: 1.5410x; 1.0125x over previous
import functools

import jax
import jax.numpy as jnp
from jax import lax
from jax.experimental import pallas as pl
from jax.experimental.pallas import tpu as pltpu
from jax.experimental.pallas import tpu_sc as plsc

BATCH = 4096
SEQ = 200
D = 64
HID = 128
CLS = 128
VOCAB = 1000000

NC = 2
NS = 16
NW = NC * NS
LANES = 16

C = 2
G = 100
GP = 128
ROWS_PER_W = BATCH // NW
CHUNKS_PER_W = ROWS_PER_W // C
NCOL = D // LANES

XB = 128


def _xpose_body(xt_ref, oa_ref, ob_ref):
    t = xt_ref[...].T
    i = pl.program_id(0)
    row = jax.lax.broadcasted_iota(jnp.int32, (XB, GP - G), 0) + i * XB
    col = jax.lax.broadcasted_iota(jnp.int32, (XB, GP - G), 1)
    dummy = (row * 797 + col * 7919) % VOCAB
    oa_ref[:, :G] = t[:, :G]
    oa_ref[:, G:] = dummy
    ob_ref[:, :G] = t[:, G:]
    ob_ref[:, G:] = dummy


@jax.jit
def _xpose(xT):
    return pl.pallas_call(
        _xpose_body,
        grid=(BATCH // XB,),
        in_specs=[pl.BlockSpec((SEQ, XB), lambda i: (0, i))],
        out_specs=[pl.BlockSpec((XB, GP), lambda i: (i, 0)),
                   pl.BlockSpec((XB, GP), lambda i: (i, 0))],
        out_shape=[jax.ShapeDtypeStruct((BATCH, GP), jnp.int32),
                   jax.ShapeDtypeStruct((BATCH, GP), jnp.int32)],
    )(xT)


def _pool_body(x2a, x2b, emb, out, idx_a, idx_b, rows_v, acc_v, sem):
    wid = lax.axis_index("s") * NC + lax.axis_index("c")

    def chunk_body(g, carry):
        chunk = wid * CHUNKS_PER_W + g
        pltpu.sync_copy(x2a.at[pl.ds(chunk * C, C)], idx_a)
        pltpu.sync_copy(x2b.at[pl.ds(chunk * C, C)], idx_b)
        cps = []
        for e in range(C):
            cps.append(pltpu.async_copy(
                emb.at[idx_a.at[e]],
                rows_v.at[pl.ds(e * 2 * GP, GP)], sem))
            cps.append(pltpu.async_copy(
                emb.at[idx_b.at[e]],
                rows_v.at[pl.ds(e * 2 * GP + GP, GP)], sem))
        for cp in cps:
            cp.wait()
        for e in range(C):
            base = e * 2 * GP

            def red_body(r, accs):
                return tuple(
                    accs[c] + rows_v[base + (c // NCOL) * GP + r,
                                     pl.ds((c % NCOL) * LANES, LANES)]
                    for c in range(2 * NCOL)
                )
            accs = lax.fori_loop(
                0, G, red_body,
                tuple(jnp.zeros((LANES,), jnp.float32)
                      for _ in range(2 * NCOL)))
            for c in range(NCOL):
                acc_v[e, pl.ds(c * LANES, LANES)] = (
                    (accs[c] + accs[NCOL + c]) * (1.0 / SEQ))
        pltpu.sync_copy(acc_v, out.at[pl.ds(chunk * C, C)])
        return carry

    lax.fori_loop(0, CHUNKS_PER_W, chunk_body, 0)


@jax.jit
def _pool(x2a, x2b, emb):
    mesh = plsc.VectorSubcoreMesh(core_axis_name="c", subcore_axis_name="s")
    return pl.kernel(
        _pool_body,
        out_type=jax.ShapeDtypeStruct((BATCH, D), jnp.float32),
        mesh=mesh,
        compiler_params=pltpu.CompilerParams(use_tc_tiling_on_sc=False),
        scratch_types=[
            pltpu.VMEM((C, GP), jnp.int32),
            pltpu.VMEM((C, GP), jnp.int32),
            pltpu.VMEM((C * 2 * GP, D), jnp.float32),
            pltpu.VMEM((C, D), jnp.float32),
            pltpu.SemaphoreType.DMA,
        ],
    )(x2a, x2b, emb)


def _mlp_body(p_ref, w1t_ref, b1_ref, w2t_ref, b2_ref, o_ref):
    h = jnp.dot(p_ref[...], w1t_ref[...],
                preferred_element_type=jnp.float32) + b1_ref[...]
    o_ref[...] = jnp.dot(h, w2t_ref[...],
                         preferred_element_type=jnp.float32) + b2_ref[...]


@jax.jit
def _mlp(pooled, W1t, b1, W2t, b2):
    return pl.pallas_call(
        _mlp_body,
        out_shape=jax.ShapeDtypeStruct((BATCH, CLS), jnp.float32),
    )(pooled, W1t, b1, W2t, b2)


def kernel(x, emb, W1, b1, W2, b2):
    x2a, x2b = _xpose(x.T)
    pooled = _pool(x2a, x2b, emb)
    return _mlp(pooled, W1.T, b1[None, :], W2.T, b2[None, :])


# speedup vs baseline: 1.8765x; 1.2177x over previous
import functools

import jax
import jax.numpy as jnp
from jax import lax
from jax.experimental import pallas as pl
from jax.experimental.pallas import tpu as pltpu
from jax.experimental.pallas import tpu_sc as plsc

BATCH = 4096
SEQ = 200
D = 64
HID = 128
CLS = 128

NC = 2
NS = 16
NW = NC * NS
LANES = 16

C = 2
G_SPLIT = ((0, 104), (104, 96))
ROWS_PER_W = BATCH // NW
CHUNKS_PER_W = ROWS_PER_W // C
NCOL = D // LANES


def _pool_body(x_hbm, emb, out, idx_all, rows0, rows1, out_stage,
               sem0, sem1):
    wid = lax.axis_index("s") * NC + lax.axis_index("c")
    base_row = wid * ROWS_PER_W

    pltpu.sync_copy(x_hbm.at[pl.ds(base_row, ROWS_PER_W)], idx_all)

    def fire(g, buf, sem):
        for e in range(C):
            b = g * C + e
            for off, size in G_SPLIT:
                pltpu.async_copy(
                    emb.at[idx_all.at[b, pl.ds(off, size)]],
                    buf.at[pl.ds(e * SEQ + off, size)], sem)

    def wait_all(buf, sem):
        pltpu.make_async_copy(emb.at[pl.ds(0, C * SEQ)], buf, sem).wait()

    def accumulate(g, buf):
        for e in range(C):
            base = e * SEQ

            def red_body(r8, accs):
                accs = list(accs)
                for rr in range(8):
                    r = r8 * 8 + rr
                    s = (rr & 1) * NCOL
                    for c in range(NCOL):
                        accs[s + c] = accs[s + c] + buf[
                            base + r, pl.ds(c * LANES, LANES)]
                return tuple(accs)

            accs = lax.fori_loop(
                0, SEQ // 8, red_body,
                tuple(jnp.zeros((LANES,), jnp.float32)
                      for _ in range(2 * NCOL)))
            row = g * C + e
            for c in range(NCOL):
                out_stage[row, pl.ds(c * LANES, LANES)] = (
                    (accs[c] + accs[NCOL + c]) * (1.0 / SEQ))

    fire(0, rows0, sem0)

    def outer(h, carry):
        ga = 2 * h
        gb = 2 * h + 1
        fire(gb, rows1, sem1)
        wait_all(rows0, sem0)
        accumulate(ga, rows0)

        @pl.when(gb + 1 < CHUNKS_PER_W)
        def _():
            fire(gb + 1, rows0, sem0)

        wait_all(rows1, sem1)
        accumulate(gb, rows1)
        return carry

    lax.fori_loop(0, CHUNKS_PER_W // 2, outer, 0)
    pltpu.sync_copy(out_stage, out.at[pl.ds(base_row, ROWS_PER_W)])


@jax.jit
def _pool(x, emb):
    mesh = plsc.VectorSubcoreMesh(core_axis_name="c", subcore_axis_name="s")
    return pl.kernel(
        _pool_body,
        out_type=jax.ShapeDtypeStruct((BATCH, D), jnp.float32),
        mesh=mesh,
        compiler_params=pltpu.CompilerParams(use_tc_tiling_on_sc=False),
        scratch_types=[
            pltpu.VMEM((ROWS_PER_W, SEQ), jnp.int32),
            pltpu.VMEM((C * SEQ, D), jnp.float32),
            pltpu.VMEM((C * SEQ, D), jnp.float32),
            pltpu.VMEM((ROWS_PER_W, D), jnp.float32),
            pltpu.SemaphoreType.DMA,
            pltpu.SemaphoreType.DMA,
        ],
    )(x, emb)


def _mlp_body(p_ref, w1t_ref, b1_ref, w2t_ref, b2_ref, o_ref):
    h = jnp.dot(p_ref[...], w1t_ref[...],
                preferred_element_type=jnp.float32) + b1_ref[...]
    o_ref[...] = jnp.dot(h, w2t_ref[...],
                         preferred_element_type=jnp.float32) + b2_ref[...]


@jax.jit
def _mlp(pooled, W1t, b1, W2t, b2):
    return pl.pallas_call(
        _mlp_body,
        out_shape=jax.ShapeDtypeStruct((BATCH, CLS), jnp.float32),
    )(pooled, W1t, b1, W2t, b2)


def kernel(x, emb, W1, b1, W2, b2):
    pooled = _pool(x, emb)
    return _mlp(pooled, W1.T, b1[None, :], W2.T, b2[None, :])


# speedup vs baseline: 1.9385x; 1.0330x over previous
import functools

import jax
import jax.numpy as jnp
from jax import lax
from jax.experimental import pallas as pl
from jax.experimental.pallas import tpu as pltpu
from jax.experimental.pallas import tpu_sc as plsc

BATCH = 4096
SEQ = 200
D = 64
HID = 128
CLS = 128

NC = 2
NS = 16
NW = NC * NS
LANES = 16

C = 2
G_SPLIT = ((0, 104), (104, 96))
ROWS_PER_W = BATCH // NW
CHUNKS_PER_W = ROWS_PER_W // C
NCOL = D // LANES


NBUF = 4


def _pool_body(x_hbm, emb, out, idx_all, rows0, rows1, rows2, rows3,
               out_stage, sem0, sem1, sem2, sem3):
    wid = lax.axis_index("s") * NC + lax.axis_index("c")
    base_row = wid * ROWS_PER_W
    bufs = (rows0, rows1, rows2, rows3)
    sems = (sem0, sem1, sem2, sem3)

    pltpu.sync_copy(x_hbm.at[pl.ds(base_row, ROWS_PER_W)], idx_all)

    def fire(g, buf, sem):
        for off, size in G_SPLIT:
            pltpu.async_copy(
                emb.at[idx_all.at[g, pl.ds(off, size)]],
                buf.at[pl.ds(off, size)], sem)

    def wait_all(buf, sem):
        pltpu.make_async_copy(emb.at[pl.ds(0, SEQ)], buf, sem).wait()

    def accumulate(g, buf):
        def red_body(r8, accs):
            accs = list(accs)
            for rr in range(8):
                r = r8 * 8 + rr
                s = (rr & 1) * NCOL
                for c in range(NCOL):
                    accs[s + c] = accs[s + c] + buf[
                        r, pl.ds(c * LANES, LANES)]
            return tuple(accs)

        accs = lax.fori_loop(
            0, SEQ // 8, red_body,
            tuple(jnp.zeros((LANES,), jnp.float32)
                  for _ in range(2 * NCOL)))
        for c in range(NCOL):
            out_stage[g, pl.ds(c * LANES, LANES)] = (
                (accs[c] + accs[NCOL + c]) * (1.0 / SEQ))

    for k in range(NBUF):
        fire(k, bufs[k], sems[k])

    def outer(h, carry):
        for k in range(NBUF):
            g = NBUF * h + k
            wait_all(bufs[k], sems[k])
            accumulate(g, bufs[k])

            @pl.when(g + NBUF < ROWS_PER_W)
            def _():
                fire(g + NBUF, bufs[k], sems[k])
        return carry

    lax.fori_loop(0, ROWS_PER_W // NBUF, outer, 0)
    pltpu.sync_copy(out_stage, out.at[pl.ds(base_row, ROWS_PER_W)])


@jax.jit
def _pool(x, emb):
    mesh = plsc.VectorSubcoreMesh(core_axis_name="c", subcore_axis_name="s")
    return pl.kernel(
        _pool_body,
        out_type=jax.ShapeDtypeStruct((BATCH, D), jnp.float32),
        mesh=mesh,
        compiler_params=pltpu.CompilerParams(use_tc_tiling_on_sc=False),
        scratch_types=[
            pltpu.VMEM((ROWS_PER_W, SEQ), jnp.int32),
            pltpu.VMEM((SEQ, D), jnp.float32),
            pltpu.VMEM((SEQ, D), jnp.float32),
            pltpu.VMEM((SEQ, D), jnp.float32),
            pltpu.VMEM((SEQ, D), jnp.float32),
            pltpu.VMEM((ROWS_PER_W, D), jnp.float32),
            pltpu.SemaphoreType.DMA,
            pltpu.SemaphoreType.DMA,
            pltpu.SemaphoreType.DMA,
            pltpu.SemaphoreType.DMA,
        ],
    )(x, emb)


def _mlp_body(p_ref, w1t_ref, b1_ref, w2t_ref, b2_ref, o_ref):
    h = jnp.dot(p_ref[...], w1t_ref[...],
                preferred_element_type=jnp.float32) + b1_ref[...]
    o_ref[...] = jnp.dot(h, w2t_ref[...],
                         preferred_element_type=jnp.float32) + b2_ref[...]


@jax.jit
def _mlp(pooled, W1t, b1, W2t, b2):
    return pl.pallas_call(
        _mlp_body,
        out_shape=jax.ShapeDtypeStruct((BATCH, CLS), jnp.float32),
    )(pooled, W1t, b1, W2t, b2)


def kernel(x, emb, W1, b1, W2, b2):
    pooled = _pool(x, emb)
    return _mlp(pooled, W1.T, b1[None, :], W2.T, b2[None, :])
